# baseline (speedup 1.0000x reference)
import functools

import jax
import jax.numpy as jnp
from jax import lax
from jax.experimental import pallas as pl
from jax.experimental.pallas import tpu as pltpu
from jax.experimental.pallas import tpu_sc as plsc

L = 16
NC, NS = 2, 16
NW = NC * NS
D = 2816
SPLIT = 2
K = 16
NBUF = 4
NACC = 8
CPB = 2
DS = D // SPLIT


def _make_kernel(B):
    bpw = B // NW
    nchunk = bpw // K
    mesh = plsc.VectorSubcoreMesh(core_axis_name="c", subcore_axis_name="s")

    @functools.partial(
        pl.kernel,
        out_type=jax.ShapeDtypeStruct((NW, L), jnp.float32),
        mesh=mesh,
        scratch_types=[
            pltpu.VMEM((bpw,), jnp.int32),
            *[pltpu.VMEM((K, DS), jnp.float32) for _ in range(NBUF)],
            pltpu.VMEM((L,), jnp.float32),
            *[pltpu.SemaphoreType.DMA for _ in range(NBUF)],
        ],
    )
    def body(idx_hbm, table_hbm, out_hbm, idx_v, *rest):
        bufs_only = rest[:NBUF]
        outv = rest[NBUF]
        sems = rest[NBUF + 1 :]
        wid = lax.axis_index("s") * NC + lax.axis_index("c")
        base = wid * bpw
        pltpu.sync_copy(idx_hbm.at[pl.ds(base, bpw)], idx_v)

        bufs = tuple(zip(bufs_only, sems))

        def issue(chunk, buf, sem):
            pltpu.async_copy(
                table_hbm.at[idx_v.at[pl.ds(chunk * K, K)]], buf, sem
            )

        for b, (buf, sem) in enumerate(bufs):
            issue(b, buf, sem)

        def accum(buf, accs):
            def col_body(j, accs):
                new = list(accs)
                for c in range(CPB):
                    for r in range(K):
                        v = buf[r, pl.ds((j * CPB + c) * L, L)]
                        new[r % NACC] = new[r % NACC] + v
                return tuple(new)

            return lax.fori_loop(0, DS // (L * CPB), col_body, accs)

        def ring_body(i, accs):
            for b, (buf, sem) in enumerate(bufs):
                pltpu.make_async_copy(
                    table_hbm.at[idx_v.at[pl.ds(0, K)]], buf, sem
                ).wait()
                accs = accum(buf, accs)
                nxt = NBUF * i + b + NBUF

                @pl.when(nxt < nchunk)
                def _():
                    issue(nxt, buf, sem)

            return accs

        accs = tuple(jnp.zeros((L,), jnp.float32) for _ in range(NACC))
        accs = lax.fori_loop(0, nchunk // NBUF, ring_body, accs)
        for b in range(nchunk % NBUF):
            buf, sem = bufs[b]
            pltpu.make_async_copy(
                table_hbm.at[idx_v.at[pl.ds(0, K)]], buf, sem
            ).wait()
            accs = accum(buf, accs)
        total = accs[0]
        for a in accs[1:]:
            total = total + a
        outv[...] = total
        pltpu.sync_copy(outv, out_hbm.at[wid])

    return body


def kernel(x, table):
    idx = jnp.reshape(x, (-1,)).astype(jnp.int32)
    idx2 = jnp.reshape(
        idx[:, None] * SPLIT + jnp.arange(SPLIT, dtype=jnp.int32), (-1,)
    )
    table2 = jnp.reshape(table, (-1, DS))
    partials = _make_kernel(idx2.shape[0])(idx2, table2)
    return jnp.sum(partials)


# speedup vs baseline: 14.3138x; 14.3138x over previous
import functools

import jax
import jax.numpy as jnp
from jax import lax
from jax.experimental import pallas as pl
from jax.experimental.pallas import tpu as pltpu
from jax.experimental.pallas import tpu_sc as plsc

L = 16
NC, NS = 2, 16
NW = NC * NS
D = 2816
K = 8
NBUF = 4
NACC = 8
CPB = 2


def _make_kernel(B):
    bpw = B // NW
    nchunk = bpw // K
    mesh = plsc.VectorSubcoreMesh(core_axis_name="c", subcore_axis_name="s")

    @functools.partial(
        pl.kernel,
        out_type=jax.ShapeDtypeStruct((NW, L), jnp.float32),
        mesh=mesh,
        scratch_types=[
            pltpu.VMEM((bpw,), jnp.int32),
            *[pltpu.VMEM((K, D), jnp.float32) for _ in range(NBUF)],
            pltpu.VMEM((L,), jnp.float32),
            *[pltpu.SemaphoreType.DMA for _ in range(NBUF)],
        ],
    )
    def body(idx_hbm, table_hbm, out_hbm, idx_v, *rest):
        bufs_only = rest[:NBUF]
        outv = rest[NBUF]
        sems = rest[NBUF + 1 :]
        wid = lax.axis_index("s") * NC + lax.axis_index("c")
        base = wid * bpw
        pltpu.sync_copy(idx_hbm.at[pl.ds(base, bpw)], idx_v)

        bufs = tuple(zip(bufs_only, sems))

        def issue(chunk, buf, sem):
            pltpu.async_copy(
                table_hbm.at[idx_v.at[pl.ds(chunk * K, K)]], buf, sem
            )

        for b, (buf, sem) in enumerate(bufs):
            issue(b, buf, sem)

        def accum(buf, accs):
            def col_body(j, accs):
                new = list(accs)
                for c in range(CPB):
                    for r in range(K):
                        v = buf[r, pl.ds((j * CPB + c) * L, L)]
                        new[r % NACC] = new[r % NACC] + v
                return tuple(new)

            return lax.fori_loop(0, D // (L * CPB), col_body, accs)

        def ring_body(i, accs):
            for b, (buf, sem) in enumerate(bufs):
                pltpu.make_async_copy(
                    table_hbm.at[idx_v.at[pl.ds(0, K)]], buf, sem
                ).wait()
                accs = accum(buf, accs)
                nxt = NBUF * i + b + NBUF

                @pl.when(nxt < nchunk)
                def _():
                    issue(nxt, buf, sem)

            return accs

        accs = tuple(jnp.zeros((L,), jnp.float32) for _ in range(NACC))
        accs = lax.fori_loop(0, nchunk // NBUF, ring_body, accs)
        for b in range(nchunk % NBUF):
            buf, sem = bufs[b]
            pltpu.make_async_copy(
                table_hbm.at[idx_v.at[pl.ds(0, K)]], buf, sem
            ).wait()
            accs = accum(buf, accs)
        total = accs[0]
        for a in accs[1:]:
            total = total + a
        outv[...] = total
        pltpu.sync_copy(outv, out_hbm.at[wid])

    return body


def kernel(x, table):
    idx = jnp.reshape(x, (-1,)).astype(jnp.int32)
    partials = _make_kernel(idx.shape[0])(idx, table)
    return jnp.sum(partials)
